# baseline (speedup 1.0000x reference)
import functools

import jax
import jax.numpy as jnp
from jax import lax
from jax.experimental import pallas as pl
from jax.experimental.pallas import tpu as pltpu
from jax.experimental.pallas import tpu_sc as plsc

_LAM = 0.9
_NSLOT = 64
_NB = 32
_NEV = 64
_N = 2 * 512 * 512
_NC = 2
_NSUB = 16
_NW = _NC * _NSUB
_COLS_W = _N // _NW
_CT = 1024
_NT = _COLS_W // _CT
_L = 16
_VPR = _CT // _L
_PAD = _NEV + _L


def _sc_combine(sev_hbm, aev_hbm, mf_hbm, x_hbm, out_hbm,
                sev_v, aev_v, w_v, c_v, acc_v, x_v, sem_mf, sem_x):
    wid = lax.axis_index("s") * _NC + lax.axis_index("c")

    pltpu.sync_copy(sev_hbm, sev_v.at[pl.ds(0, _NEV)])
    pltpu.sync_copy(aev_hbm, aev_v.at[pl.ds(0, _NEV)])

    ones = jnp.ones((_L,), jnp.float32)
    for j in range(_PAD // _L):
        c_v[pl.ds(j * _L, _L)] = ones

    lane0 = lax.iota(jnp.int32, _L) == 0
    for t in range(_NEV - 1, -1, -1):
        j, lane = divmod(t, _L)
        blk = pl.ds(j * _L, _L)
        st = sev_v[blk][lane]
        at = aev_v[blk][lane]
        v = c_v[pl.ds(st, _L)]
        sfx = v[0]
        wv = w_v[blk]
        w_v[blk] = jnp.where(lax.iota(jnp.int32, _L) == lane, at * sfx, wv)
        c_v[pl.ds(st, _L)] = jnp.where(lane0, sfx * (1.0 - at), v)

    def tile_body(g, _):
        col = wid * _COLS_W + g * _CT
        cp_mf = pltpu.async_copy(mf_hbm.at[:, pl.ds(col, _CT)], acc_v, sem_mf)
        cp_x = pltpu.async_copy(x_hbm.at[:, pl.ds(col, _CT)], x_v, sem_x)
        cp_mf.wait()
        cp_x.wait()

        def scale_body(s, _):
            cs = c_v[pl.ds(s, _L)][0]
            for v in range(_VPR):
                sl = pl.ds(v * _L, _L)
                acc_v[s, sl] = acc_v[s, sl] * cs
            return 0

        lax.fori_loop(0, _NSLOT, scale_body, 0)

        def ev_body(t, _):
            st = sev_v[pl.ds(t, _L)][0]
            wt = w_v[pl.ds(t, _L)][0]
            bt = lax.shift_right_logical(t, 1)
            for v in range(_VPR):
                sl = pl.ds(v * _L, _L)
                acc_v[st, sl] = acc_v[st, sl] + wt * x_v[bt, sl]
            return 0

        lax.fori_loop(0, _NEV, ev_body, 0)

        pltpu.sync_copy(acc_v, out_hbm.at[:, pl.ds(col, _CT)])
        return 0

    lax.fori_loop(0, _NT, tile_body, 0)


_sc_kernel = functools.partial(
    pl.kernel,
    out_type=jax.ShapeDtypeStruct((_NSLOT, _N), jnp.float32),
    mesh=plsc.VectorSubcoreMesh(core_axis_name="c", subcore_axis_name="s"),
    scratch_types=[
        pltpu.VMEM((_PAD,), jnp.int32),
        pltpu.VMEM((_PAD,), jnp.float32),
        pltpu.VMEM((_PAD,), jnp.float32),
        pltpu.VMEM((_PAD,), jnp.float32),
        pltpu.VMEM((_NSLOT, _CT), jnp.float32),
        pltpu.VMEM((_NB, _CT), jnp.float32),
        pltpu.SemaphoreType.DMA,
        pltpu.SemaphoreType.DMA,
    ],
)(_sc_combine)


def kernel(x, means_idx_0, prop_means_idx_0, mean_fields):
    b, ch, h, w = x.shape
    s = mean_fields.shape[0]
    n = ch * h * w
    xf = x.reshape(b, n)
    mf = mean_fields.reshape(s, n)
    i0 = means_idx_0.astype(jnp.int32)
    p0 = prop_means_idx_0.astype(jnp.float32)
    rate = jnp.float32(1.0 - _LAM)
    a_ev = jnp.stack([p0 * rate, (1.0 - p0) * rate], axis=1).reshape(2 * b)
    s_ev = jnp.stack([i0, i0 + 1], axis=1).reshape(2 * b)
    out = _sc_kernel(s_ev, a_ev, mf, xf)
    return out.reshape(s, ch, h, w)


# speedup vs baseline: 1.0003x; 1.0003x over previous
import functools

import jax
import jax.numpy as jnp
from jax import lax
from jax.experimental import pallas as pl
from jax.experimental.pallas import tpu as pltpu
from jax.experimental.pallas import tpu_sc as plsc

_LAM = 0.9
_NSLOT = 64
_NB = 32
_NEV = 64
_N = 2 * 512 * 512
_NC = 2
_NSUB = 16
_NW = _NC * _NSUB
_COLS_W = _N // _NW
_CT = 1024
_NT = _COLS_W // _CT
_L = 16
_VPR = _CT // _L
_PAD = _NEV + _L


def _sc_combine(sev_hbm, aev_hbm, mf_hbm, x_hbm, out_hbm,
                sev_v, aev_v, w_v, c_v, acc_v, x_v, sem_mf, sem_x):
    wid = lax.axis_index("s") * _NC + lax.axis_index("c")

    pltpu.sync_copy(sev_hbm, sev_v.at[pl.ds(0, _NEV)])
    pltpu.sync_copy(aev_hbm, aev_v.at[pl.ds(0, _NEV)])

    ones = jnp.ones((_L,), jnp.float32)
    for j in range(_PAD // _L):
        c_v[pl.ds(j * _L, _L)] = ones

    lane0 = lax.iota(jnp.int32, _L) == 0
    for t in range(_NEV - 1, -1, -1):
        j, lane = divmod(t, _L)
        blk = pl.ds(j * _L, _L)
        st = sev_v[blk][lane]
        at = aev_v[blk][lane]
        v = c_v[pl.ds(st, _L)]
        sfx = v[0]
        wv = w_v[blk]
        w_v[blk] = jnp.where(lax.iota(jnp.int32, _L) == lane, at * sfx, wv)
        c_v[pl.ds(st, _L)] = jnp.where(lane0, sfx * (1.0 - at), v)

    def tile_body(g, _):
        col = wid * _COLS_W + g * _CT
        cp_mf = pltpu.async_copy(mf_hbm.at[:, pl.ds(col, _CT)], acc_v, sem_mf)
        cp_x = pltpu.async_copy(x_hbm.at[:, pl.ds(col, _CT)], x_v, sem_x)
        cp_mf.wait()
        cp_x.wait()

        def scale_body(s, _):
            cs = c_v[pl.ds(s, _L)][0]
            for v in range(_VPR):
                sl = pl.ds(v * _L, _L)
                acc_v[s, sl] = acc_v[s, sl] * cs
            return 0

        lax.fori_loop(0, _NSLOT, scale_body, 0)

        def ev_body(t, _):
            st = sev_v[pl.ds(t, _L)][0]
            wt = w_v[pl.ds(t, _L)][0]
            bt = lax.shift_right_logical(t, 1)
            for v in range(_VPR):
                sl = pl.ds(v * _L, _L)
                acc_v[st, sl] = acc_v[st, sl] + wt * x_v[bt, sl]
            return 0

        lax.fori_loop(0, _NEV, ev_body, 0)

        pltpu.sync_copy(acc_v, out_hbm.at[:, pl.ds(col, _CT)])
        return 0

    lax.fori_loop(0, _NT, tile_body, 0)


_sc_kernel = functools.partial(
    pl.kernel,
    out_type=jax.ShapeDtypeStruct((_NSLOT, _N), jnp.float32),
    mesh=plsc.VectorSubcoreMesh(core_axis_name="c", subcore_axis_name="s"),
    scratch_types=[
        pltpu.VMEM((_PAD,), jnp.int32),
        pltpu.VMEM((_PAD,), jnp.float32),
        pltpu.VMEM((_PAD,), jnp.float32),
        pltpu.VMEM((_PAD,), jnp.float32),
        pltpu.VMEM((_NSLOT, _CT), jnp.float32),
        pltpu.VMEM((_NB, _CT), jnp.float32),
        pltpu.SemaphoreType.DMA,
        pltpu.SemaphoreType.DMA,
    ],
    compiler_params=pltpu.CompilerParams(use_tc_tiling_on_sc=True),
)(_sc_combine)


def kernel(x, means_idx_0, prop_means_idx_0, mean_fields):
    b, ch, h, w = x.shape
    s = mean_fields.shape[0]
    n = ch * h * w
    xf = x.reshape(b, n)
    mf = mean_fields.reshape(s, n)
    i0 = means_idx_0.astype(jnp.int32)
    p0 = prop_means_idx_0.astype(jnp.float32)
    rate = jnp.float32(1.0 - _LAM)
    a_ev = jnp.stack([p0 * rate, (1.0 - p0) * rate], axis=1).reshape(2 * b)
    s_ev = jnp.stack([i0, i0 + 1], axis=1).reshape(2 * b)
    out = _sc_kernel(s_ev, a_ev, mf, xf)
    return out.reshape(s, ch, h, w)
